# baseline (speedup 1.0000x reference)
import functools

import jax
import jax.numpy as jnp
from jax import lax
from jax.experimental import pallas as pl
from jax.experimental.pallas import tpu as pltpu
from jax.experimental.pallas import tpu_sc as plsc

_VOCAB = 100000
_D = 50
_DP = 64
_B = 4096
_L = 50
_H = 200
_EPS = 1e-5

_NC, _NS = 2, 16
_NW = _NC * _NS
_ROWS_W = _B // _NW
_TOK_W = _ROWS_W * _L
_CH_TOK = 2 * _L
_N_CH = _ROWS_W // 2
_GRP_TOK = 4 * _CH_TOK
_N_GRP = _N_CH // 4
_SCH = 128
_N_SCH = _TOK_W // _SCH
_HSCH = _N_SCH // 2
_HTOK = _TOK_W // 2
_TILE = 2000
_NT = _VOCAB // 2 // _TILE


def _prep_body(e2_ref, sfw_ref, sfb_ref, out_ref):
    sfw = sfw_ref[...]
    sfb = sfb_ref[...]
    e2 = e2_ref[...]
    eL = e2[:, :_D]
    eR = e2[:, _D:]
    pL = lax.dot_general(eL, sfw, (((1,), (1,)), ((), ())),
                         preferred_element_type=jnp.float32) + sfb
    pR = lax.dot_general(eR, sfw, (((1,), (1,)), ((), ())),
                         preferred_element_type=jnp.float32) + sfb
    z = jnp.zeros((_TILE, _DP - _D - 2), jnp.float32)
    out_ref[...] = jnp.concatenate([eL, pL, z, eR, pR, z], axis=1)


def _prep(embed2, sf_W, sf_b):
    return pl.pallas_call(
        _prep_body,
        grid=(_NT,),
        in_specs=[
            pl.BlockSpec((_TILE, 2 * _D), lambda i: (i, 0)),
            pl.BlockSpec((2, _D), lambda i: (0, 0)),
            pl.BlockSpec((1, 2), lambda i: (0, 0)),
        ],
        out_specs=pl.BlockSpec((_TILE, 2 * _DP), lambda i: (i, 0)),
        out_shape=jax.ShapeDtypeStruct((_VOCAB // 2, 2 * _DP), jnp.float32),
    )(embed2, sf_W, sf_b.reshape(1, 2))


def _dense_body(xs_ref, lw_ref, lb_ref, g_ref, bb_ref, fw_ref, fb_ref,
                out_ref):
    y = lax.dot_general(xs_ref[...][:, :_D], lw_ref[...],
                        (((1,), (1,)), ((), ())),
                        preferred_element_type=jnp.float32)
    y = y * (1.0 / _L) + lb_ref[...]
    mu = jnp.mean(y, axis=0, keepdims=True)
    ctr = y - mu
    var = jnp.mean(ctr * ctr, axis=0, keepdims=True)
    yh = ctr * lax.rsqrt(var + _EPS) * g_ref[...] + bb_ref[...]
    out_ref[...] = lax.dot_general(yh, fw_ref[...],
                                   (((1,), (1,)), ((), ())),
                                   preferred_element_type=jnp.float32
                                   ) + fb_ref[...]


def _dense(xsum, lin_W, lin_b, bn_gamma, bn_beta, fc_W, fc_b):
    return pl.pallas_call(
        _dense_body,
        out_shape=jax.ShapeDtypeStruct((_B, 2), jnp.float32),
    )(xsum, lin_W, lin_b.reshape(1, _H), bn_gamma.reshape(1, _H),
      bn_beta.reshape(1, _H), fc_W, fc_b.reshape(1, 2))


_MESH = plsc.VectorSubcoreMesh(core_axis_name="c", subcore_axis_name="s",
                               num_cores=_NC, num_subcores=_NS)


@functools.partial(
    pl.kernel,
    out_type=(jax.ShapeDtypeStruct((_B, _DP), jnp.float32),
              jax.ShapeDtypeStruct((_B * _L // 16, 16), jnp.float32),
              jax.ShapeDtypeStruct((_B * _L // 16, 16), jnp.float32),
              jax.ShapeDtypeStruct((_B * _L // 16, 16), jnp.float32)),
    mesh=_MESH,
    scratch_types=[
        pltpu.VMEM((_N_CH, _CH_TOK), jnp.int32),
        pltpu.VMEM((_N_SCH, _SCH), jnp.int32),
        pltpu.VMEM((_N_SCH, _SCH), jnp.int32),
        pltpu.VMEM((_GRP_TOK, _DP), jnp.float32),
        pltpu.VMEM((_GRP_TOK, _DP), jnp.float32),
        pltpu.VMEM((_HTOK, 8), jnp.float32),
        pltpu.VMEM((_TOK_W // 16, 16), jnp.float32),
        pltpu.VMEM((_TOK_W // 16, 16), jnp.float32),
        pltpu.VMEM((_HTOK // 16, 16), jnp.float32),
        pltpu.VMEM((_ROWS_W, _DP), jnp.float32),
        pltpu.SemaphoreType.DMA,
        pltpu.SemaphoreType.DMA,
        pltpu.SemaphoreType.DMA,
    ],
    compiler_params=pltpu.CompilerParams(use_tc_tiling_on_sc=False,
                                         needs_layout_passes=False),
)
def _sc_main(inpA, inpL, inpT, table, lex8,
             xsum_o, sp0_o, sp1_o, stgt_o,
             idxA, idxL, idxT, ebuf0, ebuf1, sb_t8, sb_p0, sb_p1, sb_pt,
             xacc, semP0, semP1, semT):
    wid = lax.axis_index("s") * _NC + lax.axis_index("c")
    lane = jnp.arange(16, dtype=jnp.int32)

    pltpu.sync_copy(inpA.at[wid], idxA)
    pltpu.sync_copy(inpL.at[wid], idxL)
    pltpu.sync_copy(inpT.at[wid], idxT)

    def _issue_grp(gg, ebuf, sem):
        for u in range(4):
            pltpu.async_copy(
                table.at[idxA.at[(4 * gg + u) & (_N_CH - 1)]],
                ebuf.at[pl.ds(u * _CH_TOK, _CH_TOK)], sem)

    _issue_grp(0, ebuf0, semP0)
    _issue_grp(1, ebuf1, semP1)

    def _fire_round(h):
        def _fire(c, carry):
            pltpu.async_copy(lex8.at[idxL.at[h * _HSCH + c]],
                             sb_t8.at[pl.ds(c * _SCH, _SCH)], semT)
            return carry
        lax.fori_loop(0, _HSCH, _fire, 0)

    def _repack_flush_round(h):
        def _drain(c, carry):
            pltpu.make_async_copy(lex8.at[idxL.at[0]],
                                  sb_t8.at[pl.ds(0, _SCH)], semT).wait()
            return carry
        lax.fori_loop(0, _HSCH, _drain, 0)

        def _rt(j, carry):
            tv = idxT[h * _HSCH + (j >> 3), pl.ds((16 * j) & 127, 16)]
            rows = 16 * j + lane
            v = plsc.load_gather(sb_t8, [rows, tv & 7])
            sb_pt[j, pl.ds(0, 16)] = v * 0.5 + 0.5
            return carry
        lax.fori_loop(0, _HTOK // 16, _rt, 0)

        pltpu.sync_copy(
            sb_pt, stgt_o.at[pl.ds((wid * _TOK_W + h * _HTOK) // 16,
                                   _HTOK // 16)])

    def _proc_grp(ebuf, gg):
        def _tok(r, carry):
            out = []
            for row in range(8):
                for k in range(_DP // 16):
                    v = ebuf[row * _L + r, pl.ds(k * 16, 16)]
                    out.append(carry[row * 4 + k] + v)
            return tuple(out)
        acc = lax.fori_loop(
            0, _L, _tok,
            tuple(jnp.zeros((16,), jnp.float32) for _ in range(32)))
        for row in range(8):
            for k in range(_DP // 16):
                xacc[8 * gg + row, pl.ds(k * 16, 16)] = acc[row * 4 + k]

        def _rp(q, carry):
            rows = 16 * q + lane
            v0 = plsc.load_gather(ebuf, [rows, jnp.full((16,), _D,
                                                        jnp.int32)])
            v1 = plsc.load_gather(ebuf, [rows, jnp.full((16,), _D + 1,
                                                        jnp.int32)])
            sb_p0[gg * (_GRP_TOK // 16) + q, pl.ds(0, 16)] = v0
            sb_p1[gg * (_GRP_TOK // 16) + q, pl.ds(0, 16)] = v1
            return carry
        lax.fori_loop(0, _GRP_TOK // 16, _rp, 0)

    def _pool(s, carry):
        gg0 = 2 * s
        pltpu.make_async_copy(table.at[idxA.at[0]], ebuf0, semP0).wait()
        _proc_grp(ebuf0, gg0)
        _issue_grp(gg0 + 2, ebuf0, semP0)
        pltpu.make_async_copy(table.at[idxA.at[0]], ebuf1, semP1).wait()
        _proc_grp(ebuf1, gg0 + 1)
        _issue_grp(gg0 + 3, ebuf1, semP1)
        return carry

    _fire_round(0)
    lax.fori_loop(0, _N_GRP // 4, _pool, 0)
    _repack_flush_round(0)
    _fire_round(1)
    lax.fori_loop(_N_GRP // 4, _N_GRP // 2, _pool, 0)
    _repack_flush_round(1)

    pltpu.make_async_copy(table.at[idxA.at[0]], ebuf0, semP0).wait()
    pltpu.make_async_copy(table.at[idxA.at[0]], ebuf1, semP1).wait()

    pltpu.sync_copy(xacc, xsum_o.at[pl.ds(wid * _ROWS_W, _ROWS_W)])
    pltpu.sync_copy(sb_p0, sp0_o.at[pl.ds(wid * (_TOK_W // 16),
                                          _TOK_W // 16)])
    pltpu.sync_copy(sb_p1, sp1_o.at[pl.ds(wid * (_TOK_W // 16),
                                          _TOK_W // 16)])


def kernel(inp, embed, lex, lin_W, lin_b, bn_gamma, bn_beta, fc_W, fc_b,
           sf_W, sf_b):
    inpA = inp.reshape(_NW, _N_CH, _CH_TOK)
    inpL = (inp >> 3).reshape(_NW, _N_SCH, _SCH)
    inpT = inp.reshape(_NW, _N_SCH, _SCH)
    packed = _prep(embed.reshape(_VOCAB // 2, 2 * _D), sf_W, sf_b)
    table = packed.reshape(_VOCAB, _DP)
    lex8 = lex.reshape(_VOCAB // 8, 8)
    xsum, sp0, sp1, stgt = _sc_main(inpA, inpL, inpT, table, lex8)
    output = _dense(xsum, lin_W, lin_b, bn_gamma, bn_beta, fc_W, fc_b)
    senti_output = jnp.stack(
        [sp0.reshape(_B * _L), sp1.reshape(_B * _L)], axis=-1)
    senti_target = stgt.reshape(_B * _L)
    return senti_output, senti_target, output


# speedup vs baseline: 1.2364x; 1.2364x over previous
import functools

import jax
import jax.numpy as jnp
from jax import lax
from jax.experimental import pallas as pl
from jax.experimental.pallas import tpu as pltpu
from jax.experimental.pallas import tpu_sc as plsc

_VOCAB = 100000
_D = 50
_DP = 64
_B = 4096
_L = 50
_H = 200
_EPS = 1e-5

_NC, _NS = 2, 16
_NW = _NC * _NS
_ROWS_W = _B // _NW
_TOK_W = _ROWS_W * _L
_CH_TOK = 2 * _L
_N_CH = _ROWS_W // 2
_GRP_TOK = 4 * _CH_TOK
_N_GRP = _N_CH // 4
_SCH = 128
_N_SCH = _TOK_W // _SCH
_HSCH = _N_SCH // 2
_HTOK = _TOK_W // 2
_TILE = 2000
_NT = _VOCAB // 2 // _TILE


def _phalf(e, sfw, sfb):
    p = lax.dot_general(e, sfw, (((1,), (1,)), ((), ())),
                        preferred_element_type=jnp.float32) + sfb
    return [e, p, jnp.zeros((_TILE, _DP - _D - 2), jnp.float32)]


def _prep_body(e1_ref, e2_ref, sfw_ref, sfb_ref, out_ref):
    sfw = sfw_ref[...]
    sfb = sfb_ref[...]
    out_ref[...] = jnp.concatenate(
        _phalf(e1_ref[...], sfw, sfb) + _phalf(e2_ref[...], sfw, sfb),
        axis=1)


def _prep(embed, sf_W, sf_b):
    return pl.pallas_call(
        _prep_body,
        grid=(_NT,),
        in_specs=[
            pl.BlockSpec((_TILE, _D), lambda i: (i, 0)),
            pl.BlockSpec((_TILE, _D), lambda i: (i + _NT, 0)),
            pl.BlockSpec((2, _D), lambda i: (0, 0)),
            pl.BlockSpec((1, 2), lambda i: (0, 0)),
        ],
        out_specs=pl.BlockSpec((_TILE, 2 * _DP), lambda i: (i, 0)),
        out_shape=jax.ShapeDtypeStruct((_VOCAB // 2, 2 * _DP), jnp.float32),
    )(embed, embed, sf_W, sf_b.reshape(1, 2))


def _dense_body(xs_ref, lw_ref, lb_ref, g_ref, bb_ref, fw_ref, fb_ref,
                out_ref):
    y = lax.dot_general(xs_ref[...][:, :_D], lw_ref[...],
                        (((1,), (1,)), ((), ())),
                        preferred_element_type=jnp.float32)
    y = y * (1.0 / _L) + lb_ref[...]
    mu = jnp.mean(y, axis=0, keepdims=True)
    ctr = y - mu
    var = jnp.mean(ctr * ctr, axis=0, keepdims=True)
    yh = ctr * lax.rsqrt(var + _EPS) * g_ref[...] + bb_ref[...]
    out_ref[...] = lax.dot_general(yh, fw_ref[...],
                                   (((1,), (1,)), ((), ())),
                                   preferred_element_type=jnp.float32
                                   ) + fb_ref[...]


def _dense(xsum, lin_W, lin_b, bn_gamma, bn_beta, fc_W, fc_b):
    return pl.pallas_call(
        _dense_body,
        out_shape=jax.ShapeDtypeStruct((_B, 2), jnp.float32),
    )(xsum, lin_W, lin_b.reshape(1, _H), bn_gamma.reshape(1, _H),
      bn_beta.reshape(1, _H), fc_W, fc_b.reshape(1, 2))


_MESH = plsc.VectorSubcoreMesh(core_axis_name="c", subcore_axis_name="s",
                               num_cores=_NC, num_subcores=_NS)


@functools.partial(
    pl.kernel,
    out_type=jax.ShapeDtypeStruct((_B * _L // 16, 16), jnp.float32),
    mesh=_MESH,
    scratch_types=[
        pltpu.VMEM((_N_SCH, _SCH), jnp.int32),
        pltpu.VMEM((_N_SCH, _SCH), jnp.int32),
        pltpu.VMEM((_TOK_W, 8), jnp.float32),
        pltpu.VMEM((_TOK_W // 16, 16), jnp.float32),
        pltpu.SemaphoreType.DMA,
    ],
    compiler_params=pltpu.CompilerParams(use_tc_tiling_on_sc=False,
                                         needs_layout_passes=False),
)
def _sc_tgt(inpL, inpT, lex8, stgt_o, idxL, idxT, sb_t8, sb_pt, semT):
    wid = lax.axis_index("s") * _NC + lax.axis_index("c")
    lane = jnp.arange(16, dtype=jnp.int32)

    pltpu.sync_copy(inpL.at[wid], idxL)
    pltpu.sync_copy(inpT.at[wid], idxT)

    def _fire(c, carry):
        pltpu.async_copy(lex8.at[idxL.at[c]],
                         sb_t8.at[pl.ds(c * _SCH, _SCH)], semT)
        return carry
    lax.fori_loop(0, _N_SCH, _fire, 0)

    def _drain(c, carry):
        pltpu.make_async_copy(lex8.at[idxL.at[0]],
                              sb_t8.at[pl.ds(0, _SCH)], semT).wait()
        return carry
    lax.fori_loop(0, _N_SCH, _drain, 0)

    def _rt(j, carry):
        tv = idxT[j >> 3, pl.ds((16 * j) & 127, 16)]
        rows = 16 * j + lane
        v = plsc.load_gather(sb_t8, [rows, tv & 7])
        sb_pt[j, pl.ds(0, 16)] = v * 0.5 + 0.5
        return carry
    lax.fori_loop(0, _TOK_W // 16, _rt, 0)

    pltpu.sync_copy(sb_pt,
                    stgt_o.at[pl.ds(wid * (_TOK_W // 16), _TOK_W // 16)])


@functools.partial(
    pl.kernel,
    out_type=(jax.ShapeDtypeStruct((_B, _DP), jnp.float32),
              jax.ShapeDtypeStruct((_B * _L // 16, 16), jnp.float32),
              jax.ShapeDtypeStruct((_B * _L // 16, 16), jnp.float32)),
    mesh=_MESH,
    scratch_types=[
        pltpu.VMEM((_N_CH, _CH_TOK), jnp.int32),
        pltpu.VMEM((_GRP_TOK, _DP), jnp.float32),
        pltpu.VMEM((_GRP_TOK, _DP), jnp.float32),
        pltpu.VMEM((_TOK_W // 16, 16), jnp.float32),
        pltpu.VMEM((_TOK_W // 16, 16), jnp.float32),
        pltpu.VMEM((_ROWS_W, _DP), jnp.float32),
        pltpu.SemaphoreType.DMA,
        pltpu.SemaphoreType.DMA,
    ],
    compiler_params=pltpu.CompilerParams(use_tc_tiling_on_sc=False,
                                         needs_layout_passes=False),
)
def _sc_main(inpA, table,
             xsum_o, sp0_o, sp1_o,
             idxA, ebuf0, ebuf1, sb_p0, sb_p1,
             xacc, semP0, semP1):
    wid = lax.axis_index("s") * _NC + lax.axis_index("c")
    lane = jnp.arange(16, dtype=jnp.int32)

    pltpu.sync_copy(inpA.at[wid], idxA)

    def _issue_grp(gg, ebuf, sem):
        for u in range(4):
            pltpu.async_copy(
                table.at[idxA.at[(4 * gg + u) & (_N_CH - 1)]],
                ebuf.at[pl.ds(u * _CH_TOK, _CH_TOK)], sem)

    _issue_grp(0, ebuf0, semP0)
    _issue_grp(1, ebuf1, semP1)

    def _proc_grp(ebuf, gg):
        def _tok(r, carry):
            out = list(carry)
            for t in range(2):
                for row in range(8):
                    for k in range(_DP // 16):
                        v = ebuf[row * _L + 2 * r + t, pl.ds(k * 16, 16)]
                        out[row * 4 + k] = out[row * 4 + k] + v
            return tuple(out)
        acc = lax.fori_loop(
            0, _L // 2, _tok,
            tuple(jnp.zeros((16,), jnp.float32) for _ in range(32)))
        for row in range(8):
            for k in range(_DP // 16):
                xacc[8 * gg + row, pl.ds(k * 16, 16)] = acc[row * 4 + k]

        def _rp(q, carry):
            rows = 16 * q + lane
            v0 = plsc.load_gather(ebuf, [rows, jnp.full((16,), _D,
                                                        jnp.int32)])
            v1 = plsc.load_gather(ebuf, [rows, jnp.full((16,), _D + 1,
                                                        jnp.int32)])
            sb_p0[gg * (_GRP_TOK // 16) + q, pl.ds(0, 16)] = v0
            sb_p1[gg * (_GRP_TOK // 16) + q, pl.ds(0, 16)] = v1
            return carry
        lax.fori_loop(0, _GRP_TOK // 16, _rp, 0)

    def _pool(s, carry):
        gg0 = 2 * s
        pltpu.make_async_copy(table.at[idxA.at[0]], ebuf0, semP0).wait()
        _proc_grp(ebuf0, gg0)
        _issue_grp(gg0 + 2, ebuf0, semP0)
        pltpu.make_async_copy(table.at[idxA.at[0]], ebuf1, semP1).wait()
        _proc_grp(ebuf1, gg0 + 1)
        _issue_grp(gg0 + 3, ebuf1, semP1)
        return carry

    lax.fori_loop(0, _N_GRP // 2, _pool, 0)

    pltpu.make_async_copy(table.at[idxA.at[0]], ebuf0, semP0).wait()
    pltpu.make_async_copy(table.at[idxA.at[0]], ebuf1, semP1).wait()

    pltpu.sync_copy(xacc, xsum_o.at[pl.ds(wid * _ROWS_W, _ROWS_W)])
    pltpu.sync_copy(sb_p0, sp0_o.at[pl.ds(wid * (_TOK_W // 16),
                                          _TOK_W // 16)])
    pltpu.sync_copy(sb_p1, sp1_o.at[pl.ds(wid * (_TOK_W // 16),
                                          _TOK_W // 16)])


def kernel(inp, embed, lex, lin_W, lin_b, bn_gamma, bn_beta, fc_W, fc_b,
           sf_W, sf_b):
    slot = jnp.where(inp < _VOCAB // 2, inp * 2, inp * 2 - (_VOCAB - 1))
    inpA = slot.reshape(_NW, _N_CH, _CH_TOK)
    inpL = (inp >> 3).reshape(_NW, _N_SCH, _SCH)
    inpT = inp.reshape(_NW, _N_SCH, _SCH)
    packed = _prep(embed, sf_W, sf_b)
    table = packed.reshape(_VOCAB, _DP)
    lex8 = lex.reshape(_VOCAB // 8, 8)
    stgt = _sc_tgt(inpL, inpT, lex8)
    xsum, sp0, sp1 = _sc_main(inpA, table)
    output = _dense(xsum, lin_W, lin_b, bn_gamma, bn_beta, fc_W, fc_b)
    senti_output = jnp.stack(
        [sp0.reshape(_B * _L), sp1.reshape(_B * _L)], axis=-1)
    senti_target = stgt.reshape(_B * _L)
    return senti_output, senti_target, output
